# Initial kernel scaffold; baseline (speedup 1.0000x reference)
import jax
import jax.numpy as jnp
from jax.experimental import pallas as pl


def kernel(x_u, x_v, params, edge_index_uv, edge_index_vu, edge_label_index):
    raise NotImplementedError("write your pallas kernel here")


# speedup vs baseline: 2.4114x; 2.4114x over previous
import functools
import jax
import jax.numpy as jnp
from jax import lax
from jax.experimental import pallas as pl
from jax.experimental.pallas import tpu as pltpu
from jax.experimental.pallas import tpu_sc as plsc

NN = 10000
DD = 128
EE = 160000
ELL = 50000
NSUB = 16
CH = 128

NROWS = 10240
ROWS_PER_W = NROWS // NSUB
E_PAD = 161792
EL_PAD = 51200
ECHUNKS = E_PAD // (NSUB * CH)
LCHUNKS = EL_PAD // (NSUB * CH)


@functools.lru_cache(maxsize=None)
def _sc_mesh():
    return plsc.VectorSubcoreMesh(core_axis_name="c", subcore_axis_name="s",
                                  num_cores=2, num_subcores=NSUB)


def _counts_body(dst_uv, dst_vu, zeros_hbm, ones_hbm, out,
                 acc, ones_v, idx_v, sem):
    c = lax.axis_index("c")
    s = lax.axis_index("s")
    base_r = pl.multiple_of(s * ROWS_PER_W, 8)
    pltpu.sync_copy(zeros_hbm, acc.at[pl.ds(base_r, ROWS_PER_W)])
    pltpu.sync_copy(ones_hbm, ones_v)
    plsc.subcore_barrier()

    def dir_loop(dst_ref):
        def body(j, carry):
            base = pl.multiple_of((s * ECHUNKS + j) * CH, 8)
            pltpu.sync_copy(dst_ref.at[pl.ds(base, CH)], idx_v)
            pltpu.sync_copy(ones_v, acc.at[idx_v], add=True)
            return carry
        lax.fori_loop(0, ECHUNKS, body, 0)

    @pl.when(c == 0)
    def _():
        dir_loop(dst_uv)

    @pl.when(c == 1)
    def _():
        dir_loop(dst_vu)

    plsc.subcore_barrier()
    pltpu.sync_copy(acc.at[pl.ds(base_r, ROWS_PER_W)],
                    out.at[c, pl.ds(base_r, ROWS_PER_W)])


@functools.lru_cache(maxsize=None)
def _counts_kernel():
    return pl.kernel(
        _counts_body, mesh=_sc_mesh(),
        out_type=jax.ShapeDtypeStruct((2, NROWS, DD), jnp.float32),
        scratch_types=[
            pltpu.VMEM_SHARED((NROWS, DD), jnp.float32),
            pltpu.VMEM((CH, DD), jnp.float32),
            pltpu.VMEM((CH,), jnp.int32),
            pltpu.SemaphoreType.DMA,
        ],
    )


def _agg_body(tab_u, tab_v, src_uv, dst_uv, src_vu, dst_vu, zeros_hbm, out,
              acc, rows_v, sidx_v, didx_v, sem):
    c = lax.axis_index("c")
    s = lax.axis_index("s")
    base_r = pl.multiple_of(s * ROWS_PER_W, 8)
    pltpu.sync_copy(zeros_hbm, acc.at[pl.ds(base_r, ROWS_PER_W)])
    plsc.subcore_barrier()

    def dir_loop(tab, src_ref, dst_ref):
        def body(j, carry):
            base = pl.multiple_of((s * ECHUNKS + j) * CH, 8)
            pltpu.sync_copy(src_ref.at[pl.ds(base, CH)], sidx_v)
            pltpu.sync_copy(dst_ref.at[pl.ds(base, CH)], didx_v)
            pltpu.async_copy(tab.at[sidx_v], rows_v, sem).wait()
            pltpu.sync_copy(rows_v, acc.at[didx_v], add=True)
            return carry
        lax.fori_loop(0, ECHUNKS, body, 0)

    @pl.when(c == 0)
    def _():
        dir_loop(tab_u, src_uv, dst_uv)

    @pl.when(c == 1)
    def _():
        dir_loop(tab_v, src_vu, dst_vu)

    plsc.subcore_barrier()
    pltpu.sync_copy(acc.at[pl.ds(base_r, ROWS_PER_W)],
                    out.at[c, pl.ds(base_r, ROWS_PER_W)])


@functools.lru_cache(maxsize=None)
def _agg_kernel():
    return pl.kernel(
        _agg_body, mesh=_sc_mesh(),
        out_type=jax.ShapeDtypeStruct((2, NROWS, DD), jnp.float32),
        scratch_types=[
            pltpu.VMEM_SHARED((NROWS, DD), jnp.float32),
            pltpu.VMEM((CH, DD), jnp.float32),
            pltpu.VMEM((CH,), jnp.int32),
            pltpu.VMEM((CH,), jnp.int32),
            pltpu.SemaphoreType.DMA,
        ],
    )


def _pair_body(h_u, h_v, idx0, idx1, out, rows_v, idx_v, sem):
    c = lax.axis_index("c")
    s = lax.axis_index("s")

    def dir_loop(tab, iref):
        def body(j, carry):
            base = pl.multiple_of((s * LCHUNKS + j) * CH, 8)
            pltpu.sync_copy(iref.at[pl.ds(base, CH)], idx_v)
            pltpu.async_copy(tab.at[idx_v], rows_v, sem).wait()
            pltpu.sync_copy(rows_v, out.at[c, pl.ds(base, CH)])
            return carry
        lax.fori_loop(0, LCHUNKS, body, 0)

    @pl.when(c == 0)
    def _():
        dir_loop(h_u, idx0)

    @pl.when(c == 1)
    def _():
        dir_loop(h_v, idx1)


@functools.lru_cache(maxsize=None)
def _pair_kernel():
    return pl.kernel(
        _pair_body, mesh=_sc_mesh(),
        out_type=jax.ShapeDtypeStruct((2, EL_PAD, DD), jnp.float32),
        scratch_types=[
            pltpu.VMEM((CH, DD), jnp.float32),
            pltpu.VMEM((CH,), jnp.int32),
            pltpu.SemaphoreType.DMA,
        ],
    )


RB = 1000
PB = 2048

_full = lambda shape: pl.BlockSpec(shape, lambda i: (0,) * len(shape))


def _proj_body(hu, hv, wnu, wnv, wsu, wsv, buv, bvu, pnu, pnv, psv, psu):
    hub = hu[...]
    hvb = hv[...]
    f32 = jnp.float32
    pnu[...] = jnp.dot(hub, wnu[...], preferred_element_type=f32)
    pnv[...] = jnp.dot(hvb, wnv[...], preferred_element_type=f32)
    psv[...] = jnp.dot(hvb, wsv[...], preferred_element_type=f32) + buv[...]
    psu[...] = jnp.dot(hub, wsu[...], preferred_element_type=f32) + bvu[...]


def _proj_call(hu, hv, wnu, wnv, wsu, wsv, buv, bvu):
    node = pl.BlockSpec((RB, DD), lambda i: (i, 0))
    w = _full((DD, DD))
    b = _full((1, DD))
    return pl.pallas_call(
        _proj_body,
        grid=(NN // RB,),
        in_specs=[node, node, w, w, w, w, b, b],
        out_specs=[node, node, node, node],
        out_shape=[jax.ShapeDtypeStruct((NN, DD), jnp.float32)] * 4,
    )(hu, hv, wnu, wnv, wsu, wsv, buv, bvu)


def _combine_block(ps, part, cnt):
    return jax.nn.relu(ps[...] + part[0] / jnp.maximum(cnt[0], 1.0))


def _fused_body(psu_in, psv_in, part0, part1, cnt0, cnt1,
                wnu, wnv, wsu, wsv, buv, bvu,
                hu, hv, pnu, pnv, psv, psu):
    hvb = _combine_block(psv_in, part0, cnt0)
    hub = _combine_block(psu_in, part1, cnt1)
    hu[...] = hub
    hv[...] = hvb
    f32 = jnp.float32
    pnu[...] = jnp.dot(hub, wnu[...], preferred_element_type=f32)
    pnv[...] = jnp.dot(hvb, wnv[...], preferred_element_type=f32)
    psv[...] = jnp.dot(hvb, wsv[...], preferred_element_type=f32) + buv[...]
    psu[...] = jnp.dot(hub, wsu[...], preferred_element_type=f32) + bvu[...]


def _fused_call(psu_in, psv_in, parts, cnt2, wnu, wnv, wsu, wsv, buv, bvu):
    node = pl.BlockSpec((RB, DD), lambda i: (i, 0))
    p0 = pl.BlockSpec((1, RB, DD), lambda i: (0, i, 0))
    p1 = pl.BlockSpec((1, RB, DD), lambda i: (1, i, 0))
    w = _full((DD, DD))
    b = _full((1, DD))
    return pl.pallas_call(
        _fused_body,
        grid=(NN // RB,),
        in_specs=[node, node, p0, p1, p0, p1, w, w, w, w, b, b],
        out_specs=[node] * 6,
        out_shape=[jax.ShapeDtypeStruct((NN, DD), jnp.float32)] * 6,
    )(psu_in, psv_in, parts, parts, cnt2, cnt2, wnu, wnv, wsu, wsv, buv, bvu)


def _final_comb_body(psu_in, psv_in, part0, part1, cnt0, cnt1, hu, hv):
    hv[...] = _combine_block(psv_in, part0, cnt0)
    hu[...] = _combine_block(psu_in, part1, cnt1)


def _final_comb_call(psu_in, psv_in, parts, cnt2):
    node = pl.BlockSpec((RB, DD), lambda i: (i, 0))
    p0 = pl.BlockSpec((1, RB, DD), lambda i: (0, i, 0))
    p1 = pl.BlockSpec((1, RB, DD), lambda i: (1, i, 0))
    return pl.pallas_call(
        _final_comb_body,
        grid=(NN // RB,),
        in_specs=[node, node, p0, p1, p0, p1],
        out_specs=[node, node],
        out_shape=[jax.ShapeDtypeStruct((NN, DD), jnp.float32)] * 2,
    )(psu_in, psv_in, parts, parts, cnt2, cnt2)


def _collapse_body(wsrc, bsrc, wdst, bdst, aw, av, bw, bv):
    f32 = jnp.float32

    def fold(wref, bref):
        w = wref[0]
        b = bref[0:1, :]
        for i in range(1, 10):
            w = jnp.dot(w, wref[i], preferred_element_type=f32)
            b = jnp.dot(b, wref[i], preferred_element_type=f32) + bref[i:i + 1, :]
        return w, b

    a_w, a_b = fold(wsrc, bsrc)
    b_w, b_b = fold(wdst, bdst)
    aw[...] = a_w
    av[...] = a_b
    bw[...] = b_w
    bv[...] = b_b


def _collapse_call(wsrc, bsrc, wdst, bdst):
    return pl.pallas_call(
        _collapse_body,
        grid=(1,),
        in_specs=[_full((10, DD, DD)), _full((10, DD)),
                  _full((10, DD, DD)), _full((10, DD))],
        out_specs=[_full((DD, DD)), _full((1, DD)),
                   _full((DD, DD)), _full((1, DD))],
        out_shape=[jax.ShapeDtypeStruct((DD, DD), jnp.float32),
                   jax.ShapeDtypeStruct((1, DD), jnp.float32),
                   jax.ShapeDtypeStruct((DD, DD), jnp.float32),
                   jax.ShapeDtypeStruct((1, DD), jnp.float32)],
    )(wsrc, bsrc, wdst, bdst)


def _pred_body(gs, gd, aw, av, bw, bv, out):
    f32 = jnp.float32
    s = jnp.dot(gs[0], aw[...], preferred_element_type=f32) + av[...]
    d = jnp.dot(gd[0], bw[...], preferred_element_type=f32) + bv[...]
    out[...] = jax.nn.sigmoid(jnp.sum(s * d, axis=1))


def _pred_call(g, aw, av, bw, bv):
    g0 = pl.BlockSpec((1, PB, DD), lambda i: (0, i, 0))
    g1 = pl.BlockSpec((1, PB, DD), lambda i: (1, i, 0))
    return pl.pallas_call(
        _pred_body,
        grid=(EL_PAD // PB,),
        in_specs=[g0, g1, _full((DD, DD)), _full((1, DD)),
                  _full((DD, DD)), _full((1, DD))],
        out_specs=pl.BlockSpec((PB,), lambda i: (i,)),
        out_shape=jax.ShapeDtypeStruct((EL_PAD,), jnp.float32),
    )(g, g, aw, av, bw, bv)


def _pad_edges(ei, n_pad):
    extra = n_pad - ei.shape[1]
    src = jnp.concatenate([ei[0], jnp.zeros((extra,), jnp.int32)])
    dst = jnp.concatenate([ei[1], jnp.full((extra,), NROWS - 1, jnp.int32)])
    return src, dst


def kernel(x_u, x_v, params, edge_index_uv, edge_index_vu, edge_label_index):
    src_uv, dst_uv = _pad_edges(edge_index_uv, E_PAD)
    src_vu, dst_vu = _pad_edges(edge_index_vu, E_PAD)
    extra = EL_PAD - ELL
    eli0 = jnp.concatenate([edge_label_index[0], jnp.zeros((extra,), jnp.int32)])
    eli1 = jnp.concatenate([edge_label_index[1], jnp.zeros((extra,), jnp.int32)])

    zeros_hbm = jnp.zeros((ROWS_PER_W, DD), jnp.float32)
    ones_hbm = jnp.ones((CH, DD), jnp.float32)

    cnt2 = _counts_kernel()(dst_uv, dst_vu, zeros_hbm, ones_hbm)

    def wmats(p):
        return (p["uv"]["Wneigh"], p["vu"]["Wneigh"],
                p["vu"]["Wself"], p["uv"]["Wself"],
                p["uv"]["b"].reshape(1, DD), p["vu"]["b"].reshape(1, DD))

    layers = list(params["enc"]) + list(params["dec"])

    wnu, wnv, wsu, wsv, buv, bvu = wmats(layers[0])
    pnu, pnv, psv, psu = _proj_call(x_u, x_v, wnu, wnv, wsu, wsv, buv, bvu)
    parts = _agg_kernel()(pnu, pnv, src_uv, dst_uv, src_vu, dst_vu, zeros_hbm)

    h_u = h_v = None
    for li in range(1, 5):
        wnu, wnv, wsu, wsv, buv, bvu = wmats(layers[li])
        hu, hv, pnu, pnv, psv, psu = _fused_call(
            psu, psv, parts, cnt2, wnu, wnv, wsu, wsv, buv, bvu)
        if li == 2:
            h_u, h_v = hu, hv
        parts = _agg_kernel()(pnu, pnv, src_uv, dst_uv, src_vu, dst_vu,
                              zeros_hbm)

    f_u, f_v = _final_comb_call(psu, psv, parts, cnt2)

    def stack(ps):
        ws, bs = [], []
        for p in ps:
            w, b = p["W"], p["b"]
            if w.shape[1] != DD:
                w = jnp.pad(w, ((0, 0), (0, DD - w.shape[1])))
                b = jnp.pad(b, (0, DD - b.shape[0]))
            ws.append(w)
            bs.append(b)
        return jnp.stack(ws), jnp.stack(bs)

    wsrc, bsrc = stack(params["sd_src"])
    wdst, bdst = stack(params["sd_dst"])
    aw, av, bw, bv = _collapse_call(wsrc, bsrc, wdst, bdst)

    g = _pair_kernel()(h_u, h_v, eli0, eli1)
    pred = _pred_call(g, aw, av, bw, bv)

    return (h_u, h_v, f_u, f_v, pred[:ELL])
